# Initial kernel scaffold; baseline (speedup 1.0000x reference)
import jax
import jax.numpy as jnp
from jax.experimental import pallas as pl


def kernel(node_x, nodes_map, edge_batch, edges_map, node_batch, edge_index1, edge_index2, W1, b1, W2, b2):
    raise NotImplementedError("write your pallas kernel here")


# speedup vs baseline: 1.1988x; 1.1988x over previous
import jax
import jax.numpy as jnp
from jax.experimental import pallas as pl
from jax.experimental.pallas import tpu as pltpu

N_NODES = 10000
N_HEDGES = 20000
N_INC = 320000
D = 128
N_CLASS = 40

ROW_BLK = 1000


def _segmean(data, ids, num_segments):
    sums = jax.ops.segment_sum(data, ids, num_segments=num_segments)
    cnts = jax.ops.segment_sum(jnp.ones((data.shape[0],), dtype=data.dtype),
                               ids, num_segments=num_segments)
    return sums / jnp.clip(cnts, 1.0)[:, None]


def _head_body(x0_ref, n1_ref, r2_ref, wa_ref, wbc_ref, wd_ref, b1_ref,
               w2_ref, b2_ref, out_ref):
    acc = jnp.dot(x0_ref[...], wa_ref[...], preferred_element_type=jnp.float32)
    acc += jnp.dot(n1_ref[...], wbc_ref[...], preferred_element_type=jnp.float32)
    acc += jnp.dot(r2_ref[...], wd_ref[...], preferred_element_type=jnp.float32)
    h = jax.nn.relu(acc + b1_ref[...])
    logits = jnp.dot(h, w2_ref[...], preferred_element_type=jnp.float32) + b2_ref[...]
    m = jnp.max(logits, axis=-1, keepdims=True)
    z = logits - m
    lse = jnp.log(jnp.sum(jnp.exp(z), axis=-1, keepdims=True))
    out_ref[...] = z - lse


def _head(x0, n1, r2, W1, b1, W2, b2):
    wa = W1[:D]
    wbc = W1[D:2 * D] + W1[2 * D:3 * D]
    wd = W1[3 * D:]
    grid = (N_NODES // ROW_BLK,)
    row_spec = pl.BlockSpec((ROW_BLK, D), lambda i: (i, 0))
    full = lambda shape: pl.BlockSpec(shape, lambda i: tuple(0 for _ in shape))
    return pl.pallas_call(
        _head_body,
        grid=grid,
        in_specs=[row_spec, row_spec, row_spec,
                  full((D, D)), full((D, D)), full((D, D)), full((D,)),
                  full((D, N_CLASS)), full((N_CLASS,))],
        out_specs=pl.BlockSpec((ROW_BLK, N_CLASS), lambda i: (i, 0)),
        out_shape=jax.ShapeDtypeStruct((N_NODES, N_CLASS), jnp.float32),
    )(x0, n1, r2, wa, wbc, wd, b1, W2, b2)


def kernel(node_x, nodes_map, edge_batch, edges_map, node_batch,
           edge_index1, edge_index2, W1, b1, W2, b2):
    x0 = node_x
    e1 = jax.nn.relu(_segmean(x0[nodes_map], edge_batch, N_HEDGES))
    n1 = jax.nn.relu(_segmean(e1[edges_map], node_batch, N_NODES))
    t = jax.nn.relu(_segmean(n1[nodes_map], edge_batch, N_HEDGES))
    r2 = jax.nn.relu(_segmean(t[edges_map], node_batch, N_NODES))
    return _head(x0, n1, r2, W1, b1, W2, b2)


# speedup vs baseline: 3.5426x; 2.9551x over previous
import dataclasses

import jax
from jax import lax
import jax.numpy as jnp
from jax.experimental import pallas as pl
from jax.experimental.pallas import tpu as pltpu
from jax.experimental.pallas import tpu_sc as plsc

N_NODES = 10000
N_HEDGES = 20000
N_INC = 320000
D = 128
N_CLASS = 40

NW = 32
CHUNK = N_INC // NW
WIN = 128
SB = 256
SCH = SB // 16
OB = 64
NCH = D // 16
NWIN_MAX = N_INC // (WIN - 8) + 2

ROW_BLK = 1000


def _splat_b(pred):
    return jnp.full((16,), jnp.where(pred, 1.0, 0.0), jnp.float32) > 0.5


def _sc_pass(table, map_arr, ids_arr, num_segments):
    S = num_segments
    mesh = plsc.VectorSubcoreMesh(core_axis_name="c", subcore_axis_name="s")

    def body(table_hbm, map_hbm, ids_hbm, out_hbm,
             map_v, rows_v, ids_s, search_v, ob0, ob1,
             sem_g, sem_f0, sem_f1):
        w = lax.axis_index("s") * 2 + lax.axis_index("c")

        def seg_start(p):
            nblk = (N_INC - p) // (SB - 8) + 1

            def blk(b, c):
                pos, val, done = c

                def scan_block():
                    base = jnp.minimum(
                        jnp.bitwise_and(p - 1 + b * (SB - 8), -8),
                        N_INC - SB)
                    base = pl.multiple_of(base, 8)
                    pltpu.sync_copy(ids_hbm.at[pl.ds(base, SB)],
                                    search_v.at[pl.ds(0, SB)])
                    lo = jnp.maximum(p, base + 1)

                    def chunk(cc, c2):
                        pos2, val2, done2 = c2

                        def scan_chunk():
                            o = jnp.minimum(lo - base + cc * 16, SB - 16)
                            v = search_v[pl.ds(o, 16)]
                            u = search_v[pl.ds(o - 1, 16)]
                            m = v != u
                            cnt = plsc.all_reduce_population_count(m)
                            cnt = cnt if cnt.ndim == 0 else cnt[0]
                            ffs = plsc.all_reduce_ffs(m)
                            ffs = ffs if ffs.ndim == 0 else ffs[0]

                            def found():
                                fo = o + ffs
                                fv = search_v[pl.ds(fo, 16)][0]
                                return (base + fo, fv, jnp.int32(1))

                            return lax.cond(cnt > 0, found,
                                            lambda: (pos2, val2, done2))

                        return lax.cond(done2 > 0,
                                        lambda: (pos2, val2, done2),
                                        scan_chunk)

                    return lax.fori_loop(0, SCH, chunk, (pos, val, done))

                return lax.cond(done > 0, lambda: (pos, val, done),
                                scan_block)

            pos, val, _ = lax.fori_loop(
                0, nblk, blk,
                (jnp.int32(N_INC), jnp.int32(S), jnp.int32(0)))
            return pos, val

        start0, start_val = lax.cond(
            w == 0, lambda: (jnp.int32(0), jnp.int32(0)),
            lambda: seg_start(w * CHUNK))
        end0, end_val = seg_start((w + 1) * CHUNK)
        own_lo = jnp.where(w == 0, 0, start_val)
        own_hi = end_val

        def store_row(par, row_off, vals):
            def do(buf):
                for c in range(NCH):
                    buf[pl.ds(row_off * D + c * 16, 16)] = vals[c]
            lax.cond(par == 0, lambda: do(ob0), lambda: do(ob1))

        def issue_flush(par, blk_lo):
            def do(buf, sem):
                off = pl.multiple_of(blk_lo * D, 8)
                pltpu.make_async_copy(
                    buf, out_hbm.at[pl.ds(off, OB * D)], sem).start()
            lax.cond(par == 0, lambda: do(ob0, sem_f0),
                     lambda: do(ob1, sem_f1))

        def wait_flush(par):
            def do(buf, sem):
                pltpu.make_async_copy(
                    buf, out_hbm.at[pl.ds(0, OB * D)], sem).wait()
            lax.cond(par == 0, lambda: do(ob0, sem_f0),
                     lambda: do(ob1, sem_f1))

        def advance(st, target):
            blk_lo, par, if0, if1 = st
            nadv = jnp.maximum(target - blk_lo, 0) // OB

            def step(k, c):
                bl, pr, f0, f1 = c
                issue_flush(pr, bl)
                f0 = jnp.where(pr == 0, 1, f0)
                f1 = jnp.where(pr == 1, 1, f1)
                npr = 1 - pr
                pend = jnp.where(npr == 0, f0 > 0, f1 > 0)
                lax.cond(pend, lambda: wait_flush(npr), lambda: None)
                f0 = jnp.where(jnp.logical_and(npr == 0, pend), 0, f0)
                f1 = jnp.where(jnp.logical_and(npr == 1, pend), 0, f1)
                return (bl + OB, npr, f0, f1)

            return lax.fori_loop(0, nadv, step, (blk_lo, par, if0, if1))

        zrow = [jnp.zeros((16,), jnp.float32)] * NCH

        def emit(st, row, vals):
            st = advance(st, row)
            store_row(st[1], row - st[0], vals)
            return st

        def emit_zeros(st, lo, hi):
            def zb(r, c):
                return emit(c, r, zrow)
            return lax.fori_loop(lo, hi, zb, st)

        zacc = tuple(jnp.zeros((16,), jnp.float32) for _ in range(NCH))
        st0 = (own_lo, jnp.int32(0), jnp.int32(0), jnp.int32(0))
        carry0 = (start0, jnp.int32(-1), jnp.float32(0.0), own_lo - 1,
                  st0, zacc)

        def win(k, c):
            j0, cur, n, prev, st, acc = c
            wb = jnp.minimum(jnp.bitwise_and(j0, -8), N_INC - WIN)
            wb = pl.multiple_of(wb, 8)

            def fetch():
                pltpu.sync_copy(map_hbm.at[pl.ds(wb, WIN)], map_v)
                pltpu.sync_copy(ids_hbm.at[pl.ds(wb, WIN)],
                                ids_s.at[pl.ds(0, WIN)])
                pltpu.async_copy(table_hbm.at[map_v], rows_v, sem_g).wait()
            lax.cond(j0 < end0, fetch, lambda: None)
            hi = jnp.minimum(end0, wb + WIN)

            def row_body(rr, rc):
                cur, n, prev, st, acc = rc
                off = rr - wb
                sid = ids_s[pl.ds(off, 16)][0]
                changed = sid != cur

                def flush():
                    nv = jnp.full((16,), n, jnp.float32)
                    vals = [jnp.maximum(a / nv, 0.0) for a in acc]
                    nst = emit_zeros(st, prev + 1, cur)
                    nst = emit(nst, cur, vals)
                    return nst + (cur,)

                res = lax.cond(jnp.logical_and(changed, n > 0.0), flush,
                               lambda: st + (prev,))
                nst, nprev = res[:4], res[4]

                row = tuple(rows_v[off, pl.ds(cidx * 16, 16)]
                            for cidx in range(NCH))
                chsp = _splat_b(changed)
                nacc = tuple(jnp.where(chsp, r, a + r)
                             for a, r in zip(acc, row))
                nn = jnp.where(changed, 1.0, n + 1.0)
                return (sid, nn, nprev, nst, nacc)

            cur, n, prev, st, acc = lax.fori_loop(
                j0, hi, row_body, (cur, n, prev, st, acc))
            return (hi, cur, n, prev, st, acc)

        j, cur, n, prev, st, acc = lax.fori_loop(0, NWIN_MAX, win, carry0)

        def final_flush():
            nv = jnp.full((16,), n, jnp.float32)
            vals = [jnp.maximum(a / nv, 0.0) for a in acc]
            nst = emit_zeros(st, prev + 1, cur)
            nst = emit(nst, cur, vals)
            return nst + (cur,)

        res = lax.cond(n > 0.0, final_flush, lambda: st + (prev,))
        stf, prevf = res[:4], res[4]
        stf = emit_zeros(stf, prevf + 1, own_hi)
        stf = advance(stf, own_hi)
        blk_lo, par, if0, if1 = stf

        def tail(r, _):
            def do(buf):
                soff = pl.multiple_of((r - blk_lo) * D, 8)
                doff = pl.multiple_of(r * D, 8)
                pltpu.sync_copy(buf.at[pl.ds(soff, D)],
                                out_hbm.at[pl.ds(doff, D)])
            lax.cond(par == 0, lambda: do(ob0), lambda: do(ob1))
            return 0

        lax.fori_loop(blk_lo, own_hi, tail, 0)

        lax.cond(if0 > 0, lambda: wait_flush(0), lambda: None)
        lax.cond(if1 > 0, lambda: wait_flush(1), lambda: None)

    cp = pltpu.CompilerParams()
    if "needs_layout_passes" in pltpu.CompilerParams.__dataclass_fields__:
        cp = dataclasses.replace(cp, needs_layout_passes=False)
    kern = pl.kernel(
        body,
        out_type=jax.ShapeDtypeStruct((S * D,), jnp.float32),
        mesh=mesh,
        compiler_params=cp,
        scratch_types=[
            pltpu.VMEM((WIN,), jnp.int32),
            pltpu.VMEM((WIN, D), jnp.float32),
            pltpu.VMEM((WIN + 16,), jnp.int32),
            pltpu.VMEM((SB + 16,), jnp.int32),
            pltpu.VMEM((OB * D,), jnp.float32),
            pltpu.VMEM((OB * D,), jnp.float32),
            pltpu.SemaphoreType.DMA,
            pltpu.SemaphoreType.DMA,
            pltpu.SemaphoreType.DMA,
        ],
    )
    return kern(table, map_arr, ids_arr).reshape(S, D)


def _head_body(x0_ref, n1_ref, r2_ref, wa_ref, wbc_ref, wd_ref, b1_ref,
               w2_ref, b2_ref, out_ref):
    acc = jnp.dot(x0_ref[...], wa_ref[...], preferred_element_type=jnp.float32)
    acc += jnp.dot(n1_ref[...], wbc_ref[...], preferred_element_type=jnp.float32)
    acc += jnp.dot(r2_ref[...], wd_ref[...], preferred_element_type=jnp.float32)
    h = jax.nn.relu(acc + b1_ref[...])
    logits = jnp.dot(h, w2_ref[...], preferred_element_type=jnp.float32) + b2_ref[...]
    m = jnp.max(logits, axis=-1, keepdims=True)
    z = logits - m
    lse = jnp.log(jnp.sum(jnp.exp(z), axis=-1, keepdims=True))
    out_ref[...] = z - lse


def _head(x0, n1, r2, W1, b1, W2, b2):
    wa = W1[:D]
    wbc = W1[D:2 * D] + W1[2 * D:3 * D]
    wd = W1[3 * D:]
    grid = (N_NODES // ROW_BLK,)
    row_spec = pl.BlockSpec((ROW_BLK, D), lambda i: (i, 0))
    full = lambda shape: pl.BlockSpec(shape, lambda i: tuple(0 for _ in shape))
    return pl.pallas_call(
        _head_body,
        grid=grid,
        in_specs=[row_spec, row_spec, row_spec,
                  full((D, D)), full((D, D)), full((D, D)), full((D,)),
                  full((D, N_CLASS)), full((N_CLASS,))],
        out_specs=pl.BlockSpec((ROW_BLK, N_CLASS), lambda i: (i, 0)),
        out_shape=jax.ShapeDtypeStruct((N_NODES, N_CLASS), jnp.float32),
    )(x0, n1, r2, wa, wbc, wd, b1, W2, b2)


def kernel(node_x, nodes_map, edge_batch, edges_map, node_batch,
           edge_index1, edge_index2, W1, b1, W2, b2):
    x0 = node_x
    e1 = _sc_pass(x0, nodes_map, edge_batch, N_HEDGES)
    n1 = _sc_pass(e1, edges_map, node_batch, N_NODES)
    t = _sc_pass(n1, nodes_map, edge_batch, N_HEDGES)
    r2 = _sc_pass(t, edges_map, node_batch, N_NODES)
    return _head(x0, n1, r2, W1, b1, W2, b2)


# speedup vs baseline: 6.0751x; 1.7149x over previous
import dataclasses

import jax
from jax import lax
import jax.numpy as jnp
from jax.experimental import pallas as pl
from jax.experimental.pallas import tpu as pltpu
from jax.experimental.pallas import tpu_sc as plsc

N_NODES = 10000
N_HEDGES = 20000
N_INC = 320000
D = 128
N_CLASS = 40

NW = 32
CHUNK = N_INC // NW
WIN = 128
SB = 256
SCH = SB // 16
OB = 64
NCH = D // 16
NWIN_MAX = N_INC // (WIN - 8) + 2

ROW_BLK = 1000


def _splat_b(pred):
    return jnp.full((16,), jnp.where(pred, 1.0, 0.0), jnp.float32) > 0.5


def _sc_pass(table, map_arr, ids_arr, num_segments):
    S = num_segments
    mesh = plsc.VectorSubcoreMesh(core_axis_name="c", subcore_axis_name="s")

    def body(table_hbm, map_hbm, ids_hbm, out_hbm,
             map_v0, map_v1, rows_v0, rows_v1, ids_v0, ids_v1,
             ids_cur, search_v, ob0, ob1,
             sem_m0, sem_m1, sem_g0, sem_g1, sem_f0, sem_f1):
        map_b = (map_v0, map_v1)
        rows_b = (rows_v0, rows_v1)
        ids_b = (ids_v0, ids_v1)
        sem_m = (sem_m0, sem_m1)
        sem_g = (sem_g0, sem_g1)
        w = lax.axis_index("s") * 2 + lax.axis_index("c")

        def seg_start(p):
            nblk = (N_INC - p) // (SB - 8) + 1

            def blk(b, c):
                pos, val, done = c

                def scan_block():
                    base = jnp.minimum(
                        jnp.bitwise_and(p - 1 + b * (SB - 8), -8),
                        N_INC - SB)
                    base = pl.multiple_of(base, 8)
                    pltpu.sync_copy(ids_hbm.at[pl.ds(base, SB)],
                                    search_v.at[pl.ds(0, SB)])
                    lo = jnp.maximum(p, base + 1)

                    def chunk(cc, c2):
                        pos2, val2, done2 = c2

                        def scan_chunk():
                            o = jnp.minimum(lo - base + cc * 16, SB - 16)
                            v = search_v[pl.ds(o, 16)]
                            u = search_v[pl.ds(o - 1, 16)]
                            m = v != u
                            cnt = plsc.all_reduce_population_count(m)
                            cnt = cnt if cnt.ndim == 0 else cnt[0]
                            ffs = plsc.all_reduce_ffs(m)
                            ffs = ffs if ffs.ndim == 0 else ffs[0]

                            def found():
                                fo = o + ffs
                                fv = search_v[pl.ds(fo, 16)][0]
                                return (base + fo, fv, jnp.int32(1))

                            return lax.cond(cnt > 0, found,
                                            lambda: (pos2, val2, done2))

                        return lax.cond(done2 > 0,
                                        lambda: (pos2, val2, done2),
                                        scan_chunk)

                    return lax.fori_loop(0, SCH, chunk, (pos, val, done))

                return lax.cond(done > 0, lambda: (pos, val, done),
                                scan_block)

            pos, val, _ = lax.fori_loop(
                0, nblk, blk,
                (jnp.int32(N_INC), jnp.int32(S), jnp.int32(0)))
            return pos, val

        start0, start_val = lax.cond(
            w == 0, lambda: (jnp.int32(0), jnp.int32(0)),
            lambda: seg_start(w * CHUNK))
        end0, end_val = seg_start((w + 1) * CHUNK)
        own_lo = jnp.where(w == 0, 0, start_val)
        own_hi = end_val

        def store_row(par, row_off, vals):
            def do(buf):
                for c in range(NCH):
                    buf[pl.ds(row_off * D + c * 16, 16)] = vals[c]
            lax.cond(par == 0, lambda: do(ob0), lambda: do(ob1))

        def issue_flush(par, blk_lo):
            def do(buf, sem):
                off = pl.multiple_of(blk_lo * D, 8)
                pltpu.make_async_copy(
                    buf, out_hbm.at[pl.ds(off, OB * D)], sem).start()
            lax.cond(par == 0, lambda: do(ob0, sem_f0),
                     lambda: do(ob1, sem_f1))

        def wait_flush(par):
            def do(buf, sem):
                pltpu.make_async_copy(
                    buf, out_hbm.at[pl.ds(0, OB * D)], sem).wait()
            lax.cond(par == 0, lambda: do(ob0, sem_f0),
                     lambda: do(ob1, sem_f1))

        def advance(st, target):
            blk_lo, par, if0, if1 = st
            nadv = jnp.maximum(target - blk_lo, 0) // OB

            def step(k, c):
                bl, pr, f0, f1 = c
                issue_flush(pr, bl)
                f0 = jnp.where(pr == 0, 1, f0)
                f1 = jnp.where(pr == 1, 1, f1)
                npr = 1 - pr
                pend = jnp.where(npr == 0, f0 > 0, f1 > 0)
                lax.cond(pend, lambda: wait_flush(npr), lambda: None)
                f0 = jnp.where(jnp.logical_and(npr == 0, pend), 0, f0)
                f1 = jnp.where(jnp.logical_and(npr == 1, pend), 0, f1)
                return (bl + OB, npr, f0, f1)

            return lax.fori_loop(0, nadv, step, (blk_lo, par, if0, if1))

        zrow = [jnp.zeros((16,), jnp.float32)] * NCH

        def emit(st, row, vals):
            st = advance(st, row)
            store_row(st[1], row - st[0], vals)
            return st

        def emit_zeros(st, lo, hi):
            def zb(r, c):
                return emit(c, r, zrow)
            return lax.fori_loop(lo, hi, zb, st)

        wb0 = jnp.minimum(jnp.bitwise_and(start0, -8), N_INC - WIN)
        nwin = (end0 - wb0 + WIN - 1) // WIN

        def wbase(k):
            b = jnp.minimum(wb0 + k * WIN, N_INC - WIN)
            return pl.multiple_of(b, 8)

        def fetch(k, h):
            def do():
                base = wbase(k)
                pltpu.make_async_copy(
                    map_hbm.at[pl.ds(base, WIN)], map_b[h], sem_m[h]).start()
                pltpu.make_async_copy(
                    ids_hbm.at[pl.ds(base, WIN)],
                    ids_b[h].at[pl.ds(0, WIN)], sem_m[h]).start()
            lax.cond(k < nwin, do, lambda: None)

        def gather_issue(k, h):
            def do():
                pltpu.make_async_copy(
                    map_hbm.at[pl.ds(0, WIN)], map_b[h], sem_m[h]).wait()
                pltpu.make_async_copy(
                    ids_hbm.at[pl.ds(0, WIN)],
                    ids_b[h].at[pl.ds(0, WIN)], sem_m[h]).wait()
                pltpu.make_async_copy(
                    table_hbm.at[map_b[h]], rows_b[h], sem_g[h]).start()
            lax.cond(k < nwin, do, lambda: None)

        def gwait(k, h):
            def do():
                pltpu.make_async_copy(
                    table_hbm.at[map_b[h]], rows_b[h], sem_g[h]).wait()
            lax.cond(k < nwin, do, lambda: None)

        zacc = tuple(jnp.zeros((16,), jnp.float32) for _ in range(NCH))
        st0 = (own_lo, jnp.int32(0), jnp.int32(0), jnp.int32(0))
        carry0 = (jnp.int32(-1), jnp.float32(0.0), own_lo - 1, st0, zacc)

        def process(k, h, c):
            cur, n, prev, st, acc = c
            s_k = wb0 + k * WIN
            base = wbase(k)
            lo = jnp.maximum(start0, s_k)
            hi = jnp.minimum(end0, s_k + WIN)
            rows_v = rows_b[h]
            ids_v = ids_cur

            def row_body(rr, rc):
                cur, n, prev, st, acc = rc
                off = rr - base
                sid = ids_v[pl.ds(off, 16)][0]
                changed = sid != cur

                def flush():
                    nv = jnp.full((16,), n, jnp.float32)
                    vals = [jnp.maximum(a / nv, 0.0) for a in acc]
                    nst = emit_zeros(st, prev + 1, cur)
                    nst = emit(nst, cur, vals)
                    return nst + (cur,)

                res = lax.cond(jnp.logical_and(changed, n > 0.0), flush,
                               lambda: st + (prev,))
                nst, nprev = res[:4], res[4]

                row = tuple(rows_v[off, pl.ds(cidx * 16, 16)]
                            for cidx in range(NCH))
                keepv = jnp.full((16,), jnp.where(changed, 0.0, 1.0),
                                 jnp.float32)
                nacc = tuple(a * keepv + r for a, r in zip(acc, row))
                nn = jnp.where(changed, 1.0, n + 1.0)
                return (sid, nn, nprev, nst, nacc)

            return lax.fori_loop(lo, hi, row_body, (cur, n, prev, st, acc))

        fetch(0, 0)
        gather_issue(0, 0)
        fetch(1, 1)

        def pair(m, c):
            for half in range(2):
                k = 2 * m + half
                gwait(k, half)
                gather_issue(k + 1, 1 - half)
                for cc in range(WIN // 16):
                    ids_cur[pl.ds(cc * 16, 16)] = (
                        ids_b[half][pl.ds(cc * 16, 16)])
                fetch(k + 2, half)
                c = process(k, half, c)
            return c

        cur, n, prev, st, acc = lax.fori_loop(
            0, (nwin + 1) // 2, pair, carry0)

        def final_flush():
            nv = jnp.full((16,), n, jnp.float32)
            vals = [jnp.maximum(a / nv, 0.0) for a in acc]
            nst = emit_zeros(st, prev + 1, cur)
            nst = emit(nst, cur, vals)
            return nst + (cur,)

        res = lax.cond(n > 0.0, final_flush, lambda: st + (prev,))
        stf, prevf = res[:4], res[4]
        stf = emit_zeros(stf, prevf + 1, own_hi)
        stf = advance(stf, own_hi)
        blk_lo, par, if0, if1 = stf

        def tail(r, _):
            def do(buf):
                soff = pl.multiple_of((r - blk_lo) * D, 8)
                doff = pl.multiple_of(r * D, 8)
                pltpu.sync_copy(buf.at[pl.ds(soff, D)],
                                out_hbm.at[pl.ds(doff, D)])
            lax.cond(par == 0, lambda: do(ob0), lambda: do(ob1))
            return 0

        lax.fori_loop(blk_lo, own_hi, tail, 0)

        lax.cond(if0 > 0, lambda: wait_flush(0), lambda: None)
        lax.cond(if1 > 0, lambda: wait_flush(1), lambda: None)

    cp = pltpu.CompilerParams()
    if "needs_layout_passes" in pltpu.CompilerParams.__dataclass_fields__:
        cp = dataclasses.replace(cp, needs_layout_passes=False)
    kern = pl.kernel(
        body,
        out_type=jax.ShapeDtypeStruct((S * D,), jnp.float32),
        mesh=mesh,
        compiler_params=cp,
        scratch_types=[
            pltpu.VMEM((WIN,), jnp.int32),
            pltpu.VMEM((WIN,), jnp.int32),
            pltpu.VMEM((WIN, D), jnp.float32),
            pltpu.VMEM((WIN, D), jnp.float32),
            pltpu.VMEM((WIN + 16,), jnp.int32),
            pltpu.VMEM((WIN + 16,), jnp.int32),
            pltpu.VMEM((WIN + 16,), jnp.int32),
            pltpu.VMEM((SB + 16,), jnp.int32),
            pltpu.VMEM((OB * D,), jnp.float32),
            pltpu.VMEM((OB * D,), jnp.float32),
            pltpu.SemaphoreType.DMA,
            pltpu.SemaphoreType.DMA,
            pltpu.SemaphoreType.DMA,
            pltpu.SemaphoreType.DMA,
            pltpu.SemaphoreType.DMA,
            pltpu.SemaphoreType.DMA,
        ],
    )
    return kern(table, map_arr, ids_arr).reshape(S, D)


def _head_body(x0_ref, n1_ref, r2_ref, wa_ref, wbc_ref, wd_ref, b1_ref,
               w2_ref, b2_ref, out_ref):
    acc = jnp.dot(x0_ref[...], wa_ref[...], preferred_element_type=jnp.float32)
    acc += jnp.dot(n1_ref[...], wbc_ref[...], preferred_element_type=jnp.float32)
    acc += jnp.dot(r2_ref[...], wd_ref[...], preferred_element_type=jnp.float32)
    h = jax.nn.relu(acc + b1_ref[...])
    logits = jnp.dot(h, w2_ref[...], preferred_element_type=jnp.float32) + b2_ref[...]
    m = jnp.max(logits, axis=-1, keepdims=True)
    z = logits - m
    lse = jnp.log(jnp.sum(jnp.exp(z), axis=-1, keepdims=True))
    out_ref[...] = z - lse


def _head(x0, n1, r2, W1, b1, W2, b2):
    wa = W1[:D]
    wbc = W1[D:2 * D] + W1[2 * D:3 * D]
    wd = W1[3 * D:]
    grid = (N_NODES // ROW_BLK,)
    row_spec = pl.BlockSpec((ROW_BLK, D), lambda i: (i, 0))
    full = lambda shape: pl.BlockSpec(shape, lambda i: tuple(0 for _ in shape))
    return pl.pallas_call(
        _head_body,
        grid=grid,
        in_specs=[row_spec, row_spec, row_spec,
                  full((D, D)), full((D, D)), full((D, D)), full((D,)),
                  full((D, N_CLASS)), full((N_CLASS,))],
        out_specs=pl.BlockSpec((ROW_BLK, N_CLASS), lambda i: (i, 0)),
        out_shape=jax.ShapeDtypeStruct((N_NODES, N_CLASS), jnp.float32),
    )(x0, n1, r2, wa, wbc, wd, b1, W2, b2)


def kernel(node_x, nodes_map, edge_batch, edges_map, node_batch,
           edge_index1, edge_index2, W1, b1, W2, b2):
    x0 = node_x
    e1 = _sc_pass(x0, nodes_map, edge_batch, N_HEDGES)
    n1 = _sc_pass(e1, edges_map, node_batch, N_NODES)
    t = _sc_pass(n1, nodes_map, edge_batch, N_HEDGES)
    r2 = _sc_pass(t, edges_map, node_batch, N_NODES)
    return _head(x0, n1, r2, W1, b1, W2, b2)


# speedup vs baseline: 6.1361x; 1.0100x over previous
import dataclasses

import jax
from jax import lax
import jax.numpy as jnp
from jax.experimental import pallas as pl
from jax.experimental.pallas import tpu as pltpu
from jax.experimental.pallas import tpu_sc as plsc

N_NODES = 10000
N_HEDGES = 20000
N_INC = 320000
D = 128
N_CLASS = 40

NW = 32
CHUNK = N_INC // NW
WIN = 128
SB = 256
SCH = SB // 16
OB = 64
NCH = D // 16
NWIN_MAX = N_INC // (WIN - 8) + 2

ROW_BLK = 1000


def _splat_b(pred):
    return jnp.full((16,), jnp.where(pred, 1.0, 0.0), jnp.float32) > 0.5


def _sc_pass(table, map_arr, ids_arr, num_segments):
    S = num_segments
    mesh = plsc.VectorSubcoreMesh(core_axis_name="c", subcore_axis_name="s")

    def body(table_hbm, map_hbm, ids_hbm, out_hbm,
             map_v0, map_v1, rows_v0, rows_v1, ids_v0, ids_v1,
             ids_cur, search_v, ob0, ob1,
             sem_m0, sem_m1, sem_g0, sem_g1, sem_f0, sem_f1):
        map_b = (map_v0, map_v1)
        rows_b = (rows_v0, rows_v1)
        ids_b = (ids_v0, ids_v1)
        sem_m = (sem_m0, sem_m1)
        sem_g = (sem_g0, sem_g1)
        w = lax.axis_index("s") * 2 + lax.axis_index("c")

        def seg_start(p):
            nblk = (N_INC - p) // (SB - 8) + 1

            def blk(b, c):
                pos, val, done = c

                def scan_block():
                    base = jnp.minimum(
                        jnp.bitwise_and(p - 1 + b * (SB - 8), -8),
                        N_INC - SB)
                    base = pl.multiple_of(base, 8)
                    pltpu.sync_copy(ids_hbm.at[pl.ds(base, SB)],
                                    search_v.at[pl.ds(0, SB)])
                    lo = jnp.maximum(p, base + 1)

                    def chunk(cc, c2):
                        pos2, val2, done2 = c2

                        def scan_chunk():
                            o = jnp.minimum(lo - base + cc * 16, SB - 16)
                            v = search_v[pl.ds(o, 16)]
                            u = search_v[pl.ds(o - 1, 16)]
                            m = v != u
                            cnt = plsc.all_reduce_population_count(m)
                            cnt = cnt if cnt.ndim == 0 else cnt[0]
                            ffs = plsc.all_reduce_ffs(m)
                            ffs = ffs if ffs.ndim == 0 else ffs[0]

                            def found():
                                fo = o + ffs
                                fv = search_v[pl.ds(fo, 16)][0]
                                return (base + fo, fv, jnp.int32(1))

                            return lax.cond(cnt > 0, found,
                                            lambda: (pos2, val2, done2))

                        return lax.cond(done2 > 0,
                                        lambda: (pos2, val2, done2),
                                        scan_chunk)

                    return lax.fori_loop(0, SCH, chunk, (pos, val, done))

                return lax.cond(done > 0, lambda: (pos, val, done),
                                scan_block)

            pos, val, _ = lax.fori_loop(
                0, nblk, blk,
                (jnp.int32(N_INC), jnp.int32(S), jnp.int32(0)))
            return pos, val

        start0, start_val = lax.cond(
            w == 0, lambda: (jnp.int32(0), jnp.int32(0)),
            lambda: seg_start(w * CHUNK))
        end0, end_val = seg_start((w + 1) * CHUNK)
        own_lo = jnp.where(w == 0, 0, start_val)
        own_hi = end_val

        def store_row(par, row_off, vals):
            def do(buf):
                for c in range(NCH):
                    buf[pl.ds(row_off * D + c * 16, 16)] = vals[c]
            lax.cond(par == 0, lambda: do(ob0), lambda: do(ob1))

        def issue_flush(par, blk_lo):
            def do(buf, sem):
                off = pl.multiple_of(blk_lo * D, 8)
                pltpu.make_async_copy(
                    buf, out_hbm.at[pl.ds(off, OB * D)], sem).start()
            lax.cond(par == 0, lambda: do(ob0, sem_f0),
                     lambda: do(ob1, sem_f1))

        def wait_flush(par):
            def do(buf, sem):
                pltpu.make_async_copy(
                    buf, out_hbm.at[pl.ds(0, OB * D)], sem).wait()
            lax.cond(par == 0, lambda: do(ob0, sem_f0),
                     lambda: do(ob1, sem_f1))

        def advance(st, target):
            blk_lo, par, if0, if1 = st
            nadv = jnp.maximum(target - blk_lo, 0) // OB

            def step(k, c):
                bl, pr, f0, f1 = c
                issue_flush(pr, bl)
                f0 = jnp.where(pr == 0, 1, f0)
                f1 = jnp.where(pr == 1, 1, f1)
                npr = 1 - pr
                pend = jnp.where(npr == 0, f0 > 0, f1 > 0)
                lax.cond(pend, lambda: wait_flush(npr), lambda: None)
                f0 = jnp.where(jnp.logical_and(npr == 0, pend), 0, f0)
                f1 = jnp.where(jnp.logical_and(npr == 1, pend), 0, f1)
                return (bl + OB, npr, f0, f1)

            return lax.fori_loop(0, nadv, step, (blk_lo, par, if0, if1))

        zrow = [jnp.zeros((16,), jnp.float32)] * NCH

        def emit(st, row, vals):
            st = advance(st, row)
            store_row(st[1], row - st[0], vals)
            return st

        def emit_zeros(st, lo, hi):
            def zb(r, c):
                return emit(c, r, zrow)
            return lax.fori_loop(lo, hi, zb, st)

        wb0 = jnp.minimum(jnp.bitwise_and(start0, -8), N_INC - WIN)
        nwin = (end0 - wb0 + WIN - 1) // WIN

        def wbase(k):
            b = jnp.minimum(wb0 + k * WIN, N_INC - WIN)
            return pl.multiple_of(b, 8)

        def fetch(k, h):
            def do():
                base = wbase(k)
                pltpu.make_async_copy(
                    map_hbm.at[pl.ds(base, WIN)], map_b[h], sem_m[h]).start()
                pltpu.make_async_copy(
                    ids_hbm.at[pl.ds(base, WIN)],
                    ids_b[h].at[pl.ds(0, WIN)], sem_m[h]).start()
            lax.cond(k < nwin, do, lambda: None)

        def gather_issue(k, h):
            def do():
                pltpu.make_async_copy(
                    map_hbm.at[pl.ds(0, WIN)], map_b[h], sem_m[h]).wait()
                pltpu.make_async_copy(
                    ids_hbm.at[pl.ds(0, WIN)],
                    ids_b[h].at[pl.ds(0, WIN)], sem_m[h]).wait()
                pltpu.make_async_copy(
                    table_hbm.at[map_b[h]], rows_b[h], sem_g[h]).start()
            lax.cond(k < nwin, do, lambda: None)

        def gwait(k, h):
            def do():
                pltpu.make_async_copy(
                    table_hbm.at[map_b[h]], rows_b[h], sem_g[h]).wait()
            lax.cond(k < nwin, do, lambda: None)

        zacc = tuple(jnp.zeros((16,), jnp.float32) for _ in range(NCH))
        st0 = (own_lo, jnp.int32(0), jnp.int32(0), jnp.int32(0))
        carry0 = (jnp.int32(-1), jnp.float32(0.0), own_lo - 1, st0, zacc)

        def process(k, h, c):
            cur, n, prev, st, acc = c
            s_k = wb0 + k * WIN
            base = wbase(k)
            lo = jnp.maximum(start0, s_k)
            hi = jnp.minimum(end0, s_k + WIN)
            rows_v = rows_b[h]
            ids_v = ids_cur

            def row_body(rr, rc):
                cur, n, prev, st, acc = rc
                off = rr - base
                sid = ids_v[pl.ds(off, 16)][0]
                changed = sid != cur

                def flush():
                    nv = jnp.full((16,), n, jnp.float32)
                    vals = [jnp.maximum(a / nv, 0.0) for a in acc]
                    nst = emit_zeros(st, prev + 1, cur)
                    nst = emit(nst, cur, vals)
                    return nst + (cur,)

                res = lax.cond(jnp.logical_and(changed, n > 0.0), flush,
                               lambda: st + (prev,))
                nst, nprev = res[:4], res[4]

                row = tuple(rows_v[off, pl.ds(cidx * 16, 16)]
                            for cidx in range(NCH))
                keepv = jnp.full((16,), jnp.where(changed, 0.0, 1.0),
                                 jnp.float32)
                nacc = tuple(a * keepv + r for a, r in zip(acc, row))
                nn = jnp.where(changed, 1.0, n + 1.0)
                return (sid, nn, nprev, nst, nacc)

            return lax.fori_loop(lo, hi, row_body, (cur, n, prev, st, acc))

        fetch(0, 0)
        gather_issue(0, 0)
        fetch(1, 1)

        def pair(m, c):
            for half in range(2):
                k = 2 * m + half
                gwait(k, half)
                gather_issue(k + 1, 1 - half)
                for cc in range(WIN // 16):
                    ids_cur[pl.ds(cc * 16, 16)] = (
                        ids_b[half][pl.ds(cc * 16, 16)])
                fetch(k + 2, half)
                c = process(k, half, c)
            return c

        cur, n, prev, st, acc = lax.fori_loop(
            0, (nwin + 1) // 2, pair, carry0)

        def final_flush():
            nv = jnp.full((16,), n, jnp.float32)
            vals = [jnp.maximum(a / nv, 0.0) for a in acc]
            nst = emit_zeros(st, prev + 1, cur)
            nst = emit(nst, cur, vals)
            return nst + (cur,)

        res = lax.cond(n > 0.0, final_flush, lambda: st + (prev,))
        stf, prevf = res[:4], res[4]
        stf = emit_zeros(stf, prevf + 1, own_hi)
        stf = advance(stf, own_hi)
        blk_lo, par, if0, if1 = stf

        def tail(r, _):
            def do(buf):
                soff = pl.multiple_of((r - blk_lo) * D, 8)
                doff = pl.multiple_of(r * D, 8)
                pltpu.make_async_copy(buf.at[pl.ds(soff, D)],
                                      out_hbm.at[pl.ds(doff, D)],
                                      sem_g0).start()
            lax.cond(par == 0, lambda: do(ob0), lambda: do(ob1))
            return 0

        lax.fori_loop(blk_lo, own_hi, tail, 0)

        def tail_drain(r, _):
            def do(buf):
                pltpu.make_async_copy(buf.at[pl.ds(0, D)],
                                      out_hbm.at[pl.ds(0, D)],
                                      sem_g0).wait()
            lax.cond(par == 0, lambda: do(ob0), lambda: do(ob1))
            return 0

        lax.fori_loop(blk_lo, own_hi, tail_drain, 0)

        lax.cond(if0 > 0, lambda: wait_flush(0), lambda: None)
        lax.cond(if1 > 0, lambda: wait_flush(1), lambda: None)

    cp = pltpu.CompilerParams()
    if "needs_layout_passes" in pltpu.CompilerParams.__dataclass_fields__:
        cp = dataclasses.replace(cp, needs_layout_passes=False)
    kern = pl.kernel(
        body,
        out_type=jax.ShapeDtypeStruct((S * D,), jnp.float32),
        mesh=mesh,
        compiler_params=cp,
        scratch_types=[
            pltpu.VMEM((WIN,), jnp.int32),
            pltpu.VMEM((WIN,), jnp.int32),
            pltpu.VMEM((WIN, D), jnp.float32),
            pltpu.VMEM((WIN, D), jnp.float32),
            pltpu.VMEM((WIN + 16,), jnp.int32),
            pltpu.VMEM((WIN + 16,), jnp.int32),
            pltpu.VMEM((WIN + 16,), jnp.int32),
            pltpu.VMEM((SB + 16,), jnp.int32),
            pltpu.VMEM((OB * D,), jnp.float32),
            pltpu.VMEM((OB * D,), jnp.float32),
            pltpu.SemaphoreType.DMA,
            pltpu.SemaphoreType.DMA,
            pltpu.SemaphoreType.DMA,
            pltpu.SemaphoreType.DMA,
            pltpu.SemaphoreType.DMA,
            pltpu.SemaphoreType.DMA,
        ],
    )
    return kern(table, map_arr, ids_arr).reshape(S, D)


def _head_body(x0_ref, n1_ref, r2_ref, wa_ref, wbc_ref, wd_ref, b1_ref,
               w2_ref, b2_ref, out_ref):
    acc = jnp.dot(x0_ref[...], wa_ref[...], preferred_element_type=jnp.float32)
    acc += jnp.dot(n1_ref[...], wbc_ref[...], preferred_element_type=jnp.float32)
    acc += jnp.dot(r2_ref[...], wd_ref[...], preferred_element_type=jnp.float32)
    h = jax.nn.relu(acc + b1_ref[...])
    logits = jnp.dot(h, w2_ref[...], preferred_element_type=jnp.float32) + b2_ref[...]
    m = jnp.max(logits, axis=-1, keepdims=True)
    z = logits - m
    lse = jnp.log(jnp.sum(jnp.exp(z), axis=-1, keepdims=True))
    out_ref[...] = z - lse


def _head(x0, n1, r2, W1, b1, W2, b2):
    wa = W1[:D]
    wbc = W1[D:2 * D] + W1[2 * D:3 * D]
    wd = W1[3 * D:]
    grid = (N_NODES // ROW_BLK,)
    row_spec = pl.BlockSpec((ROW_BLK, D), lambda i: (i, 0))
    full = lambda shape: pl.BlockSpec(shape, lambda i: tuple(0 for _ in shape))
    return pl.pallas_call(
        _head_body,
        grid=grid,
        in_specs=[row_spec, row_spec, row_spec,
                  full((D, D)), full((D, D)), full((D, D)), full((D,)),
                  full((D, N_CLASS)), full((N_CLASS,))],
        out_specs=pl.BlockSpec((ROW_BLK, N_CLASS), lambda i: (i, 0)),
        out_shape=jax.ShapeDtypeStruct((N_NODES, N_CLASS), jnp.float32),
    )(x0, n1, r2, wa, wbc, wd, b1, W2, b2)


def kernel(node_x, nodes_map, edge_batch, edges_map, node_batch,
           edge_index1, edge_index2, W1, b1, W2, b2):
    x0 = node_x
    e1 = _sc_pass(x0, nodes_map, edge_batch, N_HEDGES)
    n1 = _sc_pass(e1, edges_map, node_batch, N_NODES)
    t = _sc_pass(n1, nodes_map, edge_batch, N_HEDGES)
    r2 = _sc_pass(t, edges_map, node_batch, N_NODES)
    return _head(x0, n1, r2, W1, b1, W2, b2)


# speedup vs baseline: 6.9619x; 1.1346x over previous
import dataclasses

import jax
from jax import lax
import jax.numpy as jnp
from jax.experimental import pallas as pl
from jax.experimental.pallas import tpu as pltpu
from jax.experimental.pallas import tpu_sc as plsc

N_NODES = 10000
N_HEDGES = 20000
N_INC = 320000
D = 128
N_CLASS = 40

NW = 32
CHUNK = N_INC // NW
WIN = 128
SB = 256
SCH = SB // 16
OB = 64
NCH = D // 16
NWIN_MAX = N_INC // (WIN - 8) + 2

ROW_BLK = 1000


def _splat_b(pred):
    return jnp.full((16,), jnp.where(pred, 1.0, 0.0), jnp.float32) > 0.5


def _sc_pass(table, map_arr, ids_arr, num_segments):
    S = num_segments
    mesh = plsc.VectorSubcoreMesh(core_axis_name="c", subcore_axis_name="s")

    def body(table_hbm, map_hbm, ids_hbm, out_hbm,
             map_v0, map_v1, rows_v0, rows_v1, ids_v0, ids_v1,
             ids_cur, acc_v, search_v, ob0, ob1,
             sem_m0, sem_m1, sem_g0, sem_g1, sem_f0, sem_f1):
        map_b = (map_v0, map_v1)
        rows_b = (rows_v0, rows_v1)
        ids_b = (ids_v0, ids_v1)
        sem_m = (sem_m0, sem_m1)
        sem_g = (sem_g0, sem_g1)
        w = lax.axis_index("s") * 2 + lax.axis_index("c")

        def seg_start(p):
            nblk = (N_INC - p) // (SB - 8) + 1

            def blk(b, c):
                pos, val, done = c

                def scan_block():
                    base = jnp.minimum(
                        jnp.bitwise_and(p - 1 + b * (SB - 8), -8),
                        N_INC - SB)
                    base = pl.multiple_of(base, 8)
                    pltpu.sync_copy(ids_hbm.at[pl.ds(base, SB)],
                                    search_v.at[pl.ds(0, SB)])
                    lo = jnp.maximum(p, base + 1)

                    def chunk(cc, c2):
                        pos2, val2, done2 = c2

                        def scan_chunk():
                            o = jnp.minimum(lo - base + cc * 16, SB - 16)
                            v = search_v[pl.ds(o, 16)]
                            u = search_v[pl.ds(o - 1, 16)]
                            m = v != u
                            cnt = plsc.all_reduce_population_count(m)
                            cnt = cnt if cnt.ndim == 0 else cnt[0]
                            ffs = plsc.all_reduce_ffs(m)
                            ffs = ffs if ffs.ndim == 0 else ffs[0]

                            def found():
                                fo = o + ffs
                                fv = search_v[pl.ds(fo, 16)][0]
                                return (base + fo, fv, jnp.int32(1))

                            return lax.cond(cnt > 0, found,
                                            lambda: (pos2, val2, done2))

                        return lax.cond(done2 > 0,
                                        lambda: (pos2, val2, done2),
                                        scan_chunk)

                    return lax.fori_loop(0, SCH, chunk, (pos, val, done))

                return lax.cond(done > 0, lambda: (pos, val, done),
                                scan_block)

            pos, val, _ = lax.fori_loop(
                0, nblk, blk,
                (jnp.int32(N_INC), jnp.int32(S), jnp.int32(0)))
            return pos, val

        start0, start_val = lax.cond(
            w == 0, lambda: (jnp.int32(0), jnp.int32(0)),
            lambda: seg_start(w * CHUNK))
        end0, end_val = seg_start((w + 1) * CHUNK)
        own_lo = jnp.where(w == 0, 0, start_val)
        own_hi = end_val

        def store_row(par, row_off, vals):
            def do(buf):
                for c in range(NCH):
                    buf[pl.ds(row_off * D + c * 16, 16)] = vals[c]
            lax.cond(par == 0, lambda: do(ob0), lambda: do(ob1))

        def issue_flush(par, blk_lo):
            def do(buf, sem):
                off = pl.multiple_of(blk_lo * D, 8)
                pltpu.make_async_copy(
                    buf, out_hbm.at[pl.ds(off, OB * D)], sem).start()
            lax.cond(par == 0, lambda: do(ob0, sem_f0),
                     lambda: do(ob1, sem_f1))

        def wait_flush(par):
            def do(buf, sem):
                pltpu.make_async_copy(
                    buf, out_hbm.at[pl.ds(0, OB * D)], sem).wait()
            lax.cond(par == 0, lambda: do(ob0, sem_f0),
                     lambda: do(ob1, sem_f1))

        def advance(st, target):
            blk_lo, par, if0, if1 = st
            nadv = jnp.maximum(target - blk_lo, 0) // OB

            def step(k, c):
                bl, pr, f0, f1 = c
                issue_flush(pr, bl)
                f0 = jnp.where(pr == 0, 1, f0)
                f1 = jnp.where(pr == 1, 1, f1)
                npr = 1 - pr
                pend = jnp.where(npr == 0, f0 > 0, f1 > 0)
                lax.cond(pend, lambda: wait_flush(npr), lambda: None)
                f0 = jnp.where(jnp.logical_and(npr == 0, pend), 0, f0)
                f1 = jnp.where(jnp.logical_and(npr == 1, pend), 0, f1)
                return (bl + OB, npr, f0, f1)

            return lax.fori_loop(0, nadv, step, (blk_lo, par, if0, if1))

        zrow = [jnp.zeros((16,), jnp.float32)] * NCH

        def emit(st, row, vals):
            st = advance(st, row)
            store_row(st[1], row - st[0], vals)
            return st

        def emit_zeros(st, lo, hi):
            def zb(r, c):
                return emit(c, r, zrow)
            return lax.fori_loop(lo, hi, zb, st)

        wb0 = jnp.minimum(jnp.bitwise_and(start0, -8), N_INC - WIN)
        nwin = (end0 - wb0 + WIN - 1) // WIN

        def wbase(k):
            b = jnp.minimum(wb0 + k * WIN, N_INC - WIN)
            return pl.multiple_of(b, 8)

        def fetch(k, h):
            def do():
                base = wbase(k)
                pltpu.make_async_copy(
                    map_hbm.at[pl.ds(base, WIN)], map_b[h], sem_m[h]).start()
                pltpu.make_async_copy(
                    ids_hbm.at[pl.ds(base, WIN)],
                    ids_b[h].at[pl.ds(0, WIN)], sem_m[h]).start()
            lax.cond(k < nwin, do, lambda: None)

        def gather_issue(k, h):
            def do():
                pltpu.make_async_copy(
                    map_hbm.at[pl.ds(0, WIN)], map_b[h], sem_m[h]).wait()
                pltpu.make_async_copy(
                    ids_hbm.at[pl.ds(0, WIN)],
                    ids_b[h].at[pl.ds(0, WIN)], sem_m[h]).wait()
                pltpu.make_async_copy(
                    table_hbm.at[map_b[h]], rows_b[h], sem_g[h]).start()
            lax.cond(k < nwin, do, lambda: None)

        def gwait(k, h):
            def do():
                pltpu.make_async_copy(
                    table_hbm.at[map_b[h]], rows_b[h], sem_g[h]).wait()
            lax.cond(k < nwin, do, lambda: None)

        zacc = tuple(jnp.zeros((16,), jnp.float32) for _ in range(NCH))
        st0 = (own_lo, jnp.int32(0), jnp.int32(0), jnp.int32(0))
        carry0 = (jnp.int32(-1), jnp.float32(0.0), own_lo - 1, st0, zacc)

        def process(k, h, c):
            cur, n, prev, st, acc = c
            s_k = wb0 + k * WIN
            base = wbase(k)
            lo = jnp.maximum(start0, s_k)
            hi = jnp.minimum(end0, s_k + WIN)
            rows_v = rows_b[h]
            ids_v = ids_cur

            def row_body(rr, rc):
                cur, n, prev, st, acc = rc
                off = rr - base
                sid = ids_v[pl.ds(off, 16)][0]
                changed = sid != cur

                def flush():
                    nv = jnp.full((16,), n, jnp.float32)
                    vals = [jnp.maximum(a / nv, 0.0) for a in acc]
                    nst = emit_zeros(st, prev + 1, cur)
                    nst = emit(nst, cur, vals)
                    return nst + (cur,)

                res = lax.cond(jnp.logical_and(changed, n > 0.0), flush,
                               lambda: st + (prev,))
                nst, nprev = res[:4], res[4]

                row = tuple(rows_v[off, pl.ds(cidx * 16, 16)]
                            for cidx in range(NCH))
                keepv = jnp.full((16,), jnp.where(changed, 0.0, 1.0),
                                 jnp.float32)
                nacc = tuple(a * keepv + r for a, r in zip(acc, row))
                nn = jnp.where(changed, 1.0, n + 1.0)
                return (sid, nn, nprev, nst, nacc)

            glo = (lo - base + 15) // 16
            ghi = (hi - base) // 16
            pre_hi = jnp.minimum(base + glo * 16, hi)
            post_lo = jnp.maximum(base + ghi * 16, pre_hi)

            c = lax.fori_loop(lo, pre_hi, row_body, (cur, n, prev, st, acc))

            def group_body(g, gc):
                cur, n, prev, st, acc = gc
                idvec = ids_v[pl.ds(g * 16, 16)]
                m = idvec != jnp.full((16,), cur, jnp.int32)
                cnt = plsc.all_reduce_population_count(m)
                cnt = cnt if cnt.ndim == 0 else cnt[0]

                def fast():
                    a = list(acc)
                    for l in range(16):
                        for cx in range(NCH):
                            a[cx] = a[cx] + rows_v[g * 16 + l,
                                                   pl.ds(cx * 16, 16)]
                    for cx in range(NCH):
                        acc_v[pl.ds(cx * 16, 16)] = a[cx]
                    return (cur, n + 16.0, prev,
                            st[0], st[1], st[2], st[3])

                def slow():
                    r0 = base + g * 16
                    ncur, nn, nprev, nst, nacc = lax.fori_loop(
                        r0, r0 + 16, row_body, (cur, n, prev, st, acc))
                    for cx in range(NCH):
                        acc_v[pl.ds(cx * 16, 16)] = nacc[cx]
                    return (ncur, nn, nprev,
                            nst[0], nst[1], nst[2], nst[3])

                rcur, rn, rprev, s0, s1, s2, s3 = lax.cond(
                    cnt == 0, fast, slow)
                racc = tuple(acc_v[pl.ds(cx * 16, 16)]
                             for cx in range(NCH))
                return (rcur, rn, rprev, (s0, s1, s2, s3), racc)

            c = lax.fori_loop(glo, jnp.maximum(glo, ghi), group_body, c)
            cur, n, prev, st, acc = c
            return lax.fori_loop(post_lo, hi, row_body,
                                 (cur, n, prev, st, acc))

        fetch(0, 0)
        gather_issue(0, 0)
        fetch(1, 1)

        def pair(m, c):
            for half in range(2):
                k = 2 * m + half
                gwait(k, half)
                gather_issue(k + 1, 1 - half)
                for cc in range(WIN // 16):
                    ids_cur[pl.ds(cc * 16, 16)] = (
                        ids_b[half][pl.ds(cc * 16, 16)])
                fetch(k + 2, half)
                c = process(k, half, c)
            return c

        cur, n, prev, st, acc = lax.fori_loop(
            0, (nwin + 1) // 2, pair, carry0)

        def final_flush():
            nv = jnp.full((16,), n, jnp.float32)
            vals = [jnp.maximum(a / nv, 0.0) for a in acc]
            nst = emit_zeros(st, prev + 1, cur)
            nst = emit(nst, cur, vals)
            return nst + (cur,)

        res = lax.cond(n > 0.0, final_flush, lambda: st + (prev,))
        stf, prevf = res[:4], res[4]
        stf = emit_zeros(stf, prevf + 1, own_hi)
        stf = advance(stf, own_hi)
        blk_lo, par, if0, if1 = stf

        def tail(r, _):
            def do(buf):
                soff = pl.multiple_of((r - blk_lo) * D, 8)
                doff = pl.multiple_of(r * D, 8)
                pltpu.make_async_copy(buf.at[pl.ds(soff, D)],
                                      out_hbm.at[pl.ds(doff, D)],
                                      sem_g0).start()
            lax.cond(par == 0, lambda: do(ob0), lambda: do(ob1))
            return 0

        lax.fori_loop(blk_lo, own_hi, tail, 0)

        def tail_drain(r, _):
            def do(buf):
                pltpu.make_async_copy(buf.at[pl.ds(0, D)],
                                      out_hbm.at[pl.ds(0, D)],
                                      sem_g0).wait()
            lax.cond(par == 0, lambda: do(ob0), lambda: do(ob1))
            return 0

        lax.fori_loop(blk_lo, own_hi, tail_drain, 0)

        lax.cond(if0 > 0, lambda: wait_flush(0), lambda: None)
        lax.cond(if1 > 0, lambda: wait_flush(1), lambda: None)

    cp = pltpu.CompilerParams()
    if "needs_layout_passes" in pltpu.CompilerParams.__dataclass_fields__:
        cp = dataclasses.replace(cp, needs_layout_passes=False)
    kern = pl.kernel(
        body,
        out_type=jax.ShapeDtypeStruct((S * D,), jnp.float32),
        mesh=mesh,
        compiler_params=cp,
        scratch_types=[
            pltpu.VMEM((WIN,), jnp.int32),
            pltpu.VMEM((WIN,), jnp.int32),
            pltpu.VMEM((WIN, D), jnp.float32),
            pltpu.VMEM((WIN, D), jnp.float32),
            pltpu.VMEM((WIN + 16,), jnp.int32),
            pltpu.VMEM((WIN + 16,), jnp.int32),
            pltpu.VMEM((WIN + 16,), jnp.int32),
            pltpu.VMEM((D,), jnp.float32),
            pltpu.VMEM((SB + 16,), jnp.int32),
            pltpu.VMEM((OB * D,), jnp.float32),
            pltpu.VMEM((OB * D,), jnp.float32),
            pltpu.SemaphoreType.DMA,
            pltpu.SemaphoreType.DMA,
            pltpu.SemaphoreType.DMA,
            pltpu.SemaphoreType.DMA,
            pltpu.SemaphoreType.DMA,
            pltpu.SemaphoreType.DMA,
        ],
    )
    return kern(table, map_arr, ids_arr).reshape(S, D)


def _head_body(x0_ref, n1_ref, r2_ref, wa_ref, wbc_ref, wd_ref, b1_ref,
               w2_ref, b2_ref, out_ref):
    acc = jnp.dot(x0_ref[...], wa_ref[...], preferred_element_type=jnp.float32)
    acc += jnp.dot(n1_ref[...], wbc_ref[...], preferred_element_type=jnp.float32)
    acc += jnp.dot(r2_ref[...], wd_ref[...], preferred_element_type=jnp.float32)
    h = jax.nn.relu(acc + b1_ref[...])
    logits = jnp.dot(h, w2_ref[...], preferred_element_type=jnp.float32) + b2_ref[...]
    m = jnp.max(logits, axis=-1, keepdims=True)
    z = logits - m
    lse = jnp.log(jnp.sum(jnp.exp(z), axis=-1, keepdims=True))
    out_ref[...] = z - lse


def _head(x0, n1, r2, W1, b1, W2, b2):
    wa = W1[:D]
    wbc = W1[D:2 * D] + W1[2 * D:3 * D]
    wd = W1[3 * D:]
    grid = (N_NODES // ROW_BLK,)
    row_spec = pl.BlockSpec((ROW_BLK, D), lambda i: (i, 0))
    full = lambda shape: pl.BlockSpec(shape, lambda i: tuple(0 for _ in shape))
    return pl.pallas_call(
        _head_body,
        grid=grid,
        in_specs=[row_spec, row_spec, row_spec,
                  full((D, D)), full((D, D)), full((D, D)), full((D,)),
                  full((D, N_CLASS)), full((N_CLASS,))],
        out_specs=pl.BlockSpec((ROW_BLK, N_CLASS), lambda i: (i, 0)),
        out_shape=jax.ShapeDtypeStruct((N_NODES, N_CLASS), jnp.float32),
    )(x0, n1, r2, wa, wbc, wd, b1, W2, b2)


def kernel(node_x, nodes_map, edge_batch, edges_map, node_batch,
           edge_index1, edge_index2, W1, b1, W2, b2):
    x0 = node_x
    e1 = _sc_pass(x0, nodes_map, edge_batch, N_HEDGES)
    n1 = _sc_pass(e1, edges_map, node_batch, N_NODES)
    t = _sc_pass(n1, nodes_map, edge_batch, N_HEDGES)
    r2 = _sc_pass(t, edges_map, node_batch, N_NODES)
    return _head(x0, n1, r2, W1, b1, W2, b2)
